# Initial kernel scaffold; baseline (speedup 1.0000x reference)
import functools

import jax
import jax.numpy as jnp
from jax import lax
from jax.experimental import pallas as pl
from jax.experimental.pallas import tpu as pltpu
from jax.experimental.pallas import tpu_sc as plsc

_NC = 2
_NS = 16
_K = 80


def _make_sc_agg(n, e, d, with_deg):
  nw = _NC * _NS
  chunks = e // (nw * _K)
  assert chunks * nw * _K == e
  rows_pt = n // _NS
  assert rows_pt * _NS == n

  mesh = plsc.VectorSubcoreMesh(core_axis_name="c", subcore_axis_name="s")

  out_type = [jax.ShapeDtypeStruct((_NC, n, d), jnp.float32)]
  scratch = [
      pltpu.VMEM((_K,), jnp.int32),
      pltpu.VMEM((_K,), jnp.int32),
      pltpu.VMEM((_K, d), jnp.float32),
      pltpu.VMEM_SHARED((n, d), jnp.float32),
      pltpu.SemaphoreType.DMA,
  ]
  if with_deg:
    out_type.append(jax.ShapeDtypeStruct((_NC, n, 16), jnp.float32))
    scratch += [
        pltpu.VMEM((_K, 16), jnp.float32),
        pltpu.VMEM_SHARED((n, 16), jnp.float32),
    ]

  @functools.partial(
      pl.kernel, mesh=mesh, out_type=out_type, scratch_types=scratch)
  def body(table, src2, dst2, zacc, zdeg, ones, *refs):
    if with_deg:
      agg_out, deg_out, src_v, dst_v, rows_v, acc_sh, sem, ones_v, deg_sh = refs
    else:
      agg_out, src_v, dst_v, rows_v, acc_sh, sem = refs
    cid = lax.axis_index("c")
    sid = lax.axis_index("s")
    wid = cid * _NS + sid

    r0 = sid * rows_pt
    pltpu.sync_copy(zacc, acc_sh.at[pl.ds(r0, rows_pt)])
    if with_deg:
      pltpu.sync_copy(zdeg, deg_sh.at[pl.ds(r0, rows_pt)])
      pltpu.sync_copy(ones, ones_v)
    plsc.subcore_barrier()

    base = wid * chunks

    def chunk(j, carry):
      row = base + j
      pltpu.sync_copy(src2.at[row], src_v)
      pltpu.sync_copy(dst2.at[row], dst_v)
      pltpu.async_copy(table.at[src_v], rows_v, sem).wait()
      pltpu.sync_copy(rows_v, acc_sh.at[dst_v], add=True)
      if with_deg:
        pltpu.sync_copy(ones_v, deg_sh.at[dst_v], add=True)
      return carry

    lax.fori_loop(0, chunks, chunk, 0)
    plsc.subcore_barrier()

    pltpu.sync_copy(acc_sh.at[pl.ds(r0, rows_pt)],
                    agg_out.at[cid, pl.ds(r0, rows_pt)])
    if with_deg:
      pltpu.sync_copy(deg_sh.at[pl.ds(r0, rows_pt)],
                      deg_out.at[cid, pl.ds(r0, rows_pt)])

  return body


def _tc_layer1(x, agg, deg, w_self1, w_neigh1, b1, w_self2, w_neigh2):
  n, d = x.shape
  h = w_self1.shape[0]
  c = w_self2.shape[0]
  bn = 1000
  grid = (n // bn,)

  def tcb(x_b, agg_b, deg_b, ws1, wn1, b1_b, ws2, wn2,
          h1_b, h1r_b, p2_b, hs2_b):
    degs = jnp.maximum(deg_b[0, :, 0] + deg_b[1, :, 0], 1.0)
    mean = (agg_b[0] + agg_b[1]) / degs[:, None]
    dn = (((1,), (1,)), ((), ()))
    h1 = (lax.dot_general(x_b[...], ws1[...], dn,
                          preferred_element_type=jnp.float32)
          + lax.dot_general(mean, wn1[...], dn,
                            preferred_element_type=jnp.float32)
          + b1_b[...])
    h1_b[...] = h1
    h1r = jnp.maximum(h1, 0.0)
    h1r_b[...] = h1r
    p2_b[...] = lax.dot_general(h1r, wn2[...], dn,
                                preferred_element_type=jnp.float32)
    hs2_b[...] = lax.dot_general(h1r, ws2[...], dn,
                                 preferred_element_type=jnp.float32)

  return pl.pallas_call(
      tcb,
      grid=grid,
      in_specs=[
          pl.BlockSpec((bn, d), lambda i: (i, 0)),
          pl.BlockSpec((_NC, bn, d), lambda i: (0, i, 0)),
          pl.BlockSpec((_NC, bn, 16), lambda i: (0, i, 0)),
          pl.BlockSpec((h, d), lambda i: (0, 0)),
          pl.BlockSpec((h, d), lambda i: (0, 0)),
          pl.BlockSpec((1, h), lambda i: (0, 0)),
          pl.BlockSpec((c, h), lambda i: (0, 0)),
          pl.BlockSpec((c, h), lambda i: (0, 0)),
      ],
      out_specs=[
          pl.BlockSpec((bn, h), lambda i: (i, 0)),
          pl.BlockSpec((bn, h), lambda i: (i, 0)),
          pl.BlockSpec((bn, c), lambda i: (i, 0)),
          pl.BlockSpec((bn, c), lambda i: (i, 0)),
      ],
      out_shape=[
          jax.ShapeDtypeStruct((n, h), jnp.float32),
          jax.ShapeDtypeStruct((n, h), jnp.float32),
          jax.ShapeDtypeStruct((n, c), jnp.float32),
          jax.ShapeDtypeStruct((n, c), jnp.float32),
      ],
  )(x, agg, deg, w_self1, w_neigh1, b1.reshape(1, h), w_self2, w_neigh2)


def _tc_layer2(hs2, agg2, deg, b2):
  n, c = hs2.shape

  def tcc(hs2_b, agg2_b, deg_b, b2_b, h2_b):
    degs = jnp.maximum(deg_b[0, :, 0] + deg_b[1, :, 0], 1.0)
    h2_b[...] = hs2_b[...] + (agg2_b[0] + agg2_b[1]) / degs[:, None] + b2_b[...]

  return pl.pallas_call(
      tcc,
      grid=(1,),
      in_specs=[
          pl.BlockSpec((n, c), lambda i: (0, 0)),
          pl.BlockSpec((_NC, n, c), lambda i: (0, 0, 0)),
          pl.BlockSpec((_NC, n, 16), lambda i: (0, 0, 0)),
          pl.BlockSpec((1, c), lambda i: (0, 0)),
      ],
      out_specs=pl.BlockSpec((n, c), lambda i: (0, 0)),
      out_shape=jax.ShapeDtypeStruct((n, c), jnp.float32),
  )(hs2, agg2, deg, b2.reshape(1, c))


def kernel(x, edge_index, W_self1, W_neigh1, b1, W_self2, W_neigh2, b2):
  n, d = x.shape
  e = edge_index.shape[1]
  c = W_self2.shape[0]
  rows_pt = n // _NS

  src2 = edge_index[0].reshape(e // _K, _K)
  dst2 = edge_index[1].reshape(e // _K, _K)
  zacc = jnp.zeros((rows_pt, d), jnp.float32)
  zdeg = jnp.zeros((rows_pt, 16), jnp.float32)
  zacc2 = jnp.zeros((rows_pt, c), jnp.float32)
  ones = jnp.ones((_K, 16), jnp.float32)

  agg1, deg = _make_sc_agg(n, e, d, True)(x, src2, dst2, zacc, zdeg, ones)
  h1, h1r, p2, hs2 = _tc_layer1(x, agg1, deg, W_self1, W_neigh1, b1,
                                W_self2, W_neigh2)
  agg2 = _make_sc_agg(n, e, c, False)(p2, src2, dst2, zacc2, zdeg, ones)[0]
  h2 = _tc_layer2(hs2, agg2, deg, b2)
  return (h2, h1, h1r)


# speedup vs baseline: 6.1039x; 6.1039x over previous
import functools

import jax
import jax.numpy as jnp
from jax import lax
from jax.experimental import pallas as pl
from jax.experimental.pallas import tpu as pltpu
from jax.experimental.pallas import tpu_sc as plsc

_NC = 2
_NS = 16
_K = 80


def _pad_rows(n):
  q = _NS * 8
  return ((n + q - 1) // q) * q


def _make_sc_agg(n, e, d, with_deg):
  nw = _NC * _NS
  chunks = e // (nw * _K)
  assert chunks * nw * _K == e
  np_ = _pad_rows(n)
  rows_pt = np_ // _NS

  mesh = plsc.VectorSubcoreMesh(core_axis_name="c", subcore_axis_name="s")

  out_type = [jax.ShapeDtypeStruct((_NC, np_, d), jnp.float32)]
  scratch = [
      pltpu.VMEM((_K,), jnp.int32),
      pltpu.VMEM((_K,), jnp.int32),
      pltpu.VMEM((_K, d), jnp.float32),
      pltpu.VMEM_SHARED((np_, d), jnp.float32),
      pltpu.SemaphoreType.DMA,
  ]
  if with_deg:
    out_type.append(jax.ShapeDtypeStruct((_NC, np_, 16), jnp.float32))
    scratch += [
        pltpu.VMEM((_K, 16), jnp.float32),
        pltpu.VMEM_SHARED((np_, 16), jnp.float32),
    ]

  @functools.partial(
      pl.kernel, mesh=mesh, out_type=out_type, scratch_types=scratch,
      compiler_params=pltpu.CompilerParams(use_tc_tiling_on_sc=False))
  def body(table, src2, dst2, zacc, zdeg, ones, *refs):
    if with_deg:
      agg_out, deg_out, src_v, dst_v, rows_v, acc_sh, sem, ones_v, deg_sh = refs
    else:
      agg_out, src_v, dst_v, rows_v, acc_sh, sem = refs
    cid = lax.axis_index("c")
    sid = lax.axis_index("s")
    wid = cid * _NS + sid

    r0 = sid * rows_pt
    pltpu.sync_copy(zacc, acc_sh.at[pl.ds(r0, rows_pt)])
    if with_deg:
      pltpu.sync_copy(zdeg, deg_sh.at[pl.ds(r0, rows_pt)])
      pltpu.sync_copy(ones, ones_v)
    plsc.subcore_barrier()

    base = wid * chunks

    def chunk(j, carry):
      row = base + j
      pltpu.sync_copy(src2.at[row], src_v)
      pltpu.sync_copy(dst2.at[row], dst_v)
      pltpu.async_copy(table.at[src_v], rows_v, sem).wait()
      pltpu.sync_copy(rows_v, acc_sh.at[dst_v], add=True)
      if with_deg:
        pltpu.sync_copy(ones_v, deg_sh.at[dst_v], add=True)
      return carry

    lax.fori_loop(0, chunks, chunk, 0)
    plsc.subcore_barrier()

    pltpu.sync_copy(acc_sh.at[pl.ds(r0, rows_pt)],
                    agg_out.at[cid, pl.ds(r0, rows_pt)])
    if with_deg:
      pltpu.sync_copy(deg_sh.at[pl.ds(r0, rows_pt)],
                      deg_out.at[cid, pl.ds(r0, rows_pt)])

  return body


def _tc_layer1(x, agg, deg, w_self1, w_neigh1, b1, w_self2, w_neigh2):
  n, d = x.shape
  h = w_self1.shape[0]
  c = w_self2.shape[0]
  bn = 1000
  grid = (n // bn,)

  def tcb(x_b, agg_b, deg_b, ws1, wn1, b1_b, ws2, wn2,
          h1_b, h1r_b, p2_b, hs2_b):
    degs = jnp.maximum(deg_b[0, :, 0] + deg_b[1, :, 0], 1.0)
    mean = (agg_b[0] + agg_b[1]) / degs[:, None]
    dn = (((1,), (1,)), ((), ()))
    h1 = (lax.dot_general(x_b[...], ws1[...], dn,
                          preferred_element_type=jnp.float32)
          + lax.dot_general(mean, wn1[...], dn,
                            preferred_element_type=jnp.float32)
          + b1_b[...])
    h1_b[...] = h1
    h1r = jnp.maximum(h1, 0.0)
    h1r_b[...] = h1r
    p2_b[...] = lax.dot_general(h1r, wn2[...], dn,
                                preferred_element_type=jnp.float32)
    hs2_b[...] = lax.dot_general(h1r, ws2[...], dn,
                                 preferred_element_type=jnp.float32)

  return pl.pallas_call(
      tcb,
      grid=grid,
      in_specs=[
          pl.BlockSpec((bn, d), lambda i: (i, 0)),
          pl.BlockSpec((_NC, bn, d), lambda i: (0, i, 0)),
          pl.BlockSpec((_NC, bn, 16), lambda i: (0, i, 0)),
          pl.BlockSpec((h, d), lambda i: (0, 0)),
          pl.BlockSpec((h, d), lambda i: (0, 0)),
          pl.BlockSpec((1, h), lambda i: (0, 0)),
          pl.BlockSpec((c, h), lambda i: (0, 0)),
          pl.BlockSpec((c, h), lambda i: (0, 0)),
      ],
      out_specs=[
          pl.BlockSpec((bn, h), lambda i: (i, 0)),
          pl.BlockSpec((bn, h), lambda i: (i, 0)),
          pl.BlockSpec((bn, c), lambda i: (i, 0)),
          pl.BlockSpec((bn, c), lambda i: (i, 0)),
      ],
      out_shape=[
          jax.ShapeDtypeStruct((n, h), jnp.float32),
          jax.ShapeDtypeStruct((n, h), jnp.float32),
          jax.ShapeDtypeStruct((n, c), jnp.float32),
          jax.ShapeDtypeStruct((n, c), jnp.float32),
      ],
  )(x, agg, deg, w_self1, w_neigh1, b1.reshape(1, h), w_self2, w_neigh2)


def _tc_layer2(hs2, agg2, deg, b2):
  n, c = hs2.shape

  def tcc(hs2_b, agg2_b, deg_b, b2_b, h2_b):
    degs = jnp.maximum(deg_b[0, :, 0] + deg_b[1, :, 0], 1.0)
    h2_b[...] = hs2_b[...] + (agg2_b[0] + agg2_b[1]) / degs[:, None] + b2_b[...]

  return pl.pallas_call(
      tcc,
      grid=(1,),
      in_specs=[
          pl.BlockSpec((n, c), lambda i: (0, 0)),
          pl.BlockSpec((_NC, n, c), lambda i: (0, 0, 0)),
          pl.BlockSpec((_NC, n, 16), lambda i: (0, 0, 0)),
          pl.BlockSpec((1, c), lambda i: (0, 0)),
      ],
      out_specs=pl.BlockSpec((n, c), lambda i: (0, 0)),
      out_shape=jax.ShapeDtypeStruct((n, c), jnp.float32),
  )(hs2, agg2, deg, b2.reshape(1, c))


def kernel(x, edge_index, W_self1, W_neigh1, b1, W_self2, W_neigh2, b2):
  n, d = x.shape
  e = edge_index.shape[1]
  c = W_self2.shape[0]
  rows_pt = _pad_rows(n) // _NS

  src2 = edge_index[0].reshape(e // _K, _K)
  dst2 = edge_index[1].reshape(e // _K, _K)
  zacc = jnp.zeros((rows_pt, d), jnp.float32)
  zdeg = jnp.zeros((rows_pt, 16), jnp.float32)
  zacc2 = jnp.zeros((rows_pt, c), jnp.float32)
  ones = jnp.ones((_K, 16), jnp.float32)

  agg1, deg = _make_sc_agg(n, e, d, True)(x, src2, dst2, zacc, zdeg, ones)
  h1, h1r, p2, hs2 = _tc_layer1(x, agg1, deg, W_self1, W_neigh1, b1,
                                W_self2, W_neigh2)
  agg2 = _make_sc_agg(n, e, c, False)(p2, src2, dst2, zacc2, zdeg, ones)[0]
  h2 = _tc_layer2(hs2, agg2, deg, b2)
  return (h2, h1, h1r)


# speedup vs baseline: 13.9141x; 2.2795x over previous
import functools

import jax
import jax.numpy as jnp
from jax import lax
from jax.experimental import pallas as pl
from jax.experimental.pallas import tpu as pltpu
from jax.experimental.pallas import tpu_sc as plsc

_NC = 2
_NS = 16
_K = 80
_DW = 8


def _pad_rows(n):
  q = _NS * 8
  return ((n + q - 1) // q) * q


def _make_sc_agg(n, e, d, with_deg):
  nw = _NC * _NS
  chunks = e // (nw * _K)
  assert chunks * nw * _K == e
  np_ = _pad_rows(n)
  rows_pt = np_ // _NS

  mesh = plsc.VectorSubcoreMesh(core_axis_name="c", subcore_axis_name="s")

  out_type = [jax.ShapeDtypeStruct((_NC, np_, d), jnp.float32)]
  scratch = [
      pltpu.VMEM((chunks, _K), jnp.int32),
      pltpu.VMEM((chunks, _K), jnp.int32),
      pltpu.VMEM((_K, d), jnp.float32),
      pltpu.VMEM((_K, d), jnp.float32),
      pltpu.VMEM_SHARED((np_, d), jnp.float32),
      pltpu.SemaphoreType.DMA,
      pltpu.SemaphoreType.DMA,
  ]
  if with_deg:
    out_type.append(jax.ShapeDtypeStruct((_NC, np_, _DW), jnp.float32))
    scratch += [
        pltpu.VMEM((_K, _DW), jnp.float32),
        pltpu.VMEM_SHARED((np_, _DW), jnp.float32),
    ]

  @functools.partial(
      pl.kernel, mesh=mesh, out_type=out_type, scratch_types=scratch,
      compiler_params=pltpu.CompilerParams(use_tc_tiling_on_sc=False))
  def body(table, src2, dst2, zacc, zdeg, ones, *refs):
    if with_deg:
      (agg_out, deg_out, srcs_v, dsts_v, rows0_v, rows1_v, acc_sh, sem0, sem1,
       ones_v, deg_sh) = refs
    else:
      agg_out, srcs_v, dsts_v, rows0_v, rows1_v, acc_sh, sem0, sem1 = refs
    cid = lax.axis_index("c")
    sid = lax.axis_index("s")
    wid = cid * _NS + sid

    r0 = sid * rows_pt
    base = wid * chunks
    pltpu.sync_copy(src2.at[pl.ds(base, chunks)], srcs_v)
    pltpu.sync_copy(dst2.at[pl.ds(base, chunks)], dsts_v)
    pltpu.sync_copy(zacc, acc_sh.at[pl.ds(r0, rows_pt)])
    if with_deg:
      pltpu.sync_copy(zdeg, deg_sh.at[pl.ds(r0, rows_pt)])
      pltpu.sync_copy(ones, ones_v)
    plsc.subcore_barrier()

    def gather(j, rows_v, sem):
      pltpu.make_async_copy(table.at[srcs_v.at[j]], rows_v, sem).start()

    def wait_scatter(j, rows_v, sem):
      pltpu.make_async_copy(table.at[srcs_v.at[j]], rows_v, sem).wait()
      pltpu.sync_copy(rows_v, acc_sh.at[dsts_v.at[j]], add=True)
      if with_deg:
        pltpu.sync_copy(ones_v, deg_sh.at[dsts_v.at[j]], add=True)

    gather(0, rows0_v, sem0)

    def pair(i, carry):
      j0 = 2 * i
      gather(j0 + 1, rows1_v, sem1)
      wait_scatter(j0, rows0_v, sem0)

      @pl.when(j0 + 2 < chunks)
      def _():
        gather(j0 + 2, rows0_v, sem0)

      wait_scatter(j0 + 1, rows1_v, sem1)
      return carry

    lax.fori_loop(0, chunks // 2, pair, 0)
    if chunks % 2:
      wait_scatter(chunks - 1, rows0_v, sem0)
    plsc.subcore_barrier()

    pltpu.sync_copy(acc_sh.at[pl.ds(r0, rows_pt)],
                    agg_out.at[cid, pl.ds(r0, rows_pt)])
    if with_deg:
      pltpu.sync_copy(deg_sh.at[pl.ds(r0, rows_pt)],
                      deg_out.at[cid, pl.ds(r0, rows_pt)])

  return body


def _tc_layer1(x, agg, deg, w_self1, w_neigh1, b1, w_self2, w_neigh2):
  n, d = x.shape
  h = w_self1.shape[0]
  c = w_self2.shape[0]
  bn = 1000
  grid = (n // bn,)

  def tcb(x_b, agg_b, deg_b, ws1, wn1, b1_b, ws2, wn2,
          h1_b, h1r_b, p2_b, hs2_b):
    degs = jnp.maximum(deg_b[0, :, 0] + deg_b[1, :, 0], 1.0)
    mean = (agg_b[0] + agg_b[1]) / degs[:, None]
    dn = (((1,), (1,)), ((), ()))
    h1 = (lax.dot_general(x_b[...], ws1[...], dn,
                          preferred_element_type=jnp.float32)
          + lax.dot_general(mean, wn1[...], dn,
                            preferred_element_type=jnp.float32)
          + b1_b[...])
    h1_b[...] = h1
    h1r = jnp.maximum(h1, 0.0)
    h1r_b[...] = h1r
    p2_b[...] = lax.dot_general(h1r, wn2[...], dn,
                                preferred_element_type=jnp.float32)
    hs2_b[...] = lax.dot_general(h1r, ws2[...], dn,
                                 preferred_element_type=jnp.float32)

  return pl.pallas_call(
      tcb,
      grid=grid,
      in_specs=[
          pl.BlockSpec((bn, d), lambda i: (i, 0)),
          pl.BlockSpec((_NC, bn, d), lambda i: (0, i, 0)),
          pl.BlockSpec((_NC, bn, _DW), lambda i: (0, i, 0)),
          pl.BlockSpec((h, d), lambda i: (0, 0)),
          pl.BlockSpec((h, d), lambda i: (0, 0)),
          pl.BlockSpec((1, h), lambda i: (0, 0)),
          pl.BlockSpec((c, h), lambda i: (0, 0)),
          pl.BlockSpec((c, h), lambda i: (0, 0)),
      ],
      out_specs=[
          pl.BlockSpec((bn, h), lambda i: (i, 0)),
          pl.BlockSpec((bn, h), lambda i: (i, 0)),
          pl.BlockSpec((bn, c), lambda i: (i, 0)),
          pl.BlockSpec((bn, c), lambda i: (i, 0)),
      ],
      out_shape=[
          jax.ShapeDtypeStruct((n, h), jnp.float32),
          jax.ShapeDtypeStruct((n, h), jnp.float32),
          jax.ShapeDtypeStruct((n, c), jnp.float32),
          jax.ShapeDtypeStruct((n, c), jnp.float32),
      ],
  )(x, agg, deg, w_self1, w_neigh1, b1.reshape(1, h), w_self2, w_neigh2)


def _tc_layer2(hs2, agg2, deg, b2):
  n, c = hs2.shape

  def tcc(hs2_b, agg2_b, deg_b, b2_b, h2_b):
    degs = jnp.maximum(deg_b[0, :, 0] + deg_b[1, :, 0], 1.0)
    h2_b[...] = hs2_b[...] + (agg2_b[0] + agg2_b[1]) / degs[:, None] + b2_b[...]

  return pl.pallas_call(
      tcc,
      grid=(1,),
      in_specs=[
          pl.BlockSpec((n, c), lambda i: (0, 0)),
          pl.BlockSpec((_NC, n, c), lambda i: (0, 0, 0)),
          pl.BlockSpec((_NC, n, _DW), lambda i: (0, 0, 0)),
          pl.BlockSpec((1, c), lambda i: (0, 0)),
      ],
      out_specs=pl.BlockSpec((n, c), lambda i: (0, 0)),
      out_shape=jax.ShapeDtypeStruct((n, c), jnp.float32),
  )(hs2, agg2, deg, b2.reshape(1, c))


def kernel(x, edge_index, W_self1, W_neigh1, b1, W_self2, W_neigh2, b2):
  n, d = x.shape
  e = edge_index.shape[1]
  c = W_self2.shape[0]
  rows_pt = _pad_rows(n) // _NS

  src2 = edge_index[0].reshape(e // _K, _K)
  dst2 = edge_index[1].reshape(e // _K, _K)
  zacc = jnp.zeros((rows_pt, d), jnp.float32)
  zdeg = jnp.zeros((rows_pt, _DW), jnp.float32)
  zacc2 = jnp.zeros((rows_pt, c), jnp.float32)
  ones = jnp.ones((_K, _DW), jnp.float32)

  agg1, deg = _make_sc_agg(n, e, d, True)(x, src2, dst2, zacc, zdeg, ones)
  h1, h1r, p2, hs2 = _tc_layer1(x, agg1, deg, W_self1, W_neigh1, b1,
                                W_self2, W_neigh2)
  agg2 = _make_sc_agg(n, e, c, False)(p2, src2, dst2, zacc2, zdeg, ones)[0]
  h2 = _tc_layer2(hs2, agg2, deg, b2)
  return (h2, h1, h1r)
